# Initial kernel scaffold; baseline (speedup 1.0000x reference)
import jax
import jax.numpy as jnp
from jax.experimental import pallas as pl


def kernel(x, edge_index, We1, be1, We2, be2, W0, b0, W1, b1, W2, b2):
    raise NotImplementedError("write your pallas kernel here")


# speedup vs baseline: 4.5643x; 4.5643x over previous
import functools

import jax
import jax.numpy as jnp
from jax import lax
from jax.experimental import pallas as pl
from jax.experimental.pallas import tpu as pltpu
from jax.experimental.pallas import tpu_sc as plsc

N = 100000
E = 1600000
F = 64
NC, NS, LN = 2, 16, 16

CHUNK = 128
NCHUNKS = E // CHUNK

RANGE = 26624
NPAD = 4 * RANGE
STRIPE = RANGE // NS
ACC_ROWS = RANGE + 8

SEG_BASE = NCHUNKS // NS
SEG_EXTRA = NCHUNKS - NS * SEG_BASE

DEG_PER_SC = NCHUNKS // NC
DEG_BASE = DEG_PER_SC // NS
DEG_EXTRA = DEG_PER_SC - NS * DEG_BASE
DEG_STRIPE = 6256
NH = NS * DEG_STRIPE

_MESH = plsc.VectorSubcoreMesh(core_axis_name="c", subcore_axis_name="s")
_SC_PARAMS = pltpu.CompilerParams(use_tc_tiling_on_sc=False)


def _deg_sc(dst, ones_tile, zeros_deg):
    @functools.partial(
        pl.kernel,
        out_type=jax.ShapeDtypeStruct((NC, NH, LN), jnp.float32),
        mesh=_MESH,
        compiler_params=_SC_PARAMS,
        scratch_types=[
            pltpu.VMEM((CHUNK,), jnp.int32),
            pltpu.VMEM((CHUNK, LN), jnp.float32),
            pltpu.VMEM_SHARED((NH, LN), jnp.float32),
        ],
    )
    def k(dst_hbm, ones_hbm, zeros_hbm, out_hbm, dvec, ones_v, acc):
        cid = lax.axis_index("c")
        sid = lax.axis_index("s")
        pltpu.sync_copy(ones_hbm, ones_v)
        pltpu.sync_copy(zeros_hbm, acc.at[pl.ds(sid * DEG_STRIPE, DEG_STRIPE)])
        plsc.subcore_barrier()

        def hist(g):
            pltpu.sync_copy(dst_hbm.at[pl.ds(g * CHUNK, CHUNK)], dvec)
            pltpu.sync_copy(ones_v, acc.at[dvec], add=True)

        @pl.loop(0, DEG_BASE)
        def _(i):
            hist(cid * DEG_PER_SC + sid * DEG_BASE + i)

        @pl.when(sid < DEG_EXTRA)
        def _():
            hist(cid * DEG_PER_SC + NS * DEG_BASE + sid)

        plsc.subcore_barrier()
        pltpu.sync_copy(
            acc.at[pl.ds(sid * DEG_STRIPE, DEG_STRIPE)],
            out_hbm.at[cid, pl.ds(sid * DEG_STRIPE, DEG_STRIPE)],
        )

    return k(dst, ones_tile, zeros_deg)


def _segsum_sc(xws, src, dst, zeros_acc):
    @functools.partial(
        pl.kernel,
        out_type=jax.ShapeDtypeStruct((NPAD, F), jnp.float32),
        mesh=_MESH,
        compiler_params=_SC_PARAMS,
        scratch_types=[
            pltpu.VMEM((CHUNK,), jnp.int32),
            pltpu.VMEM((CHUNK,), jnp.int32),
            pltpu.VMEM((CHUNK,), jnp.int32),
            pltpu.VMEM((CHUNK, F), jnp.float32),
            pltpu.VMEM_SHARED((ACC_ROWS, F), jnp.float32),
        ],
    )
    def k(xws_hbm, src_hbm, dst_hbm, zeros_hbm, out_hbm, svec, dvec, lidx, rows, acc):
        cid = lax.axis_index("c")
        sid = lax.axis_index("s")

        for p in (0, 1):
            r = 2 * cid + p
            lo = r * RANGE
            lo_v = jnp.full((LN,), lo, jnp.int32)
            hi_v = jnp.full((LN,), lo + RANGE, jnp.int32)

            pltpu.sync_copy(zeros_hbm, acc.at[pl.ds(sid * STRIPE, STRIPE)])

            @pl.when(sid == 0)
            def _():
                pltpu.sync_copy(zeros_hbm.at[pl.ds(0, 8)], acc.at[pl.ds(RANGE, 8)])

            plsc.subcore_barrier()

            def chunk(g):
                base = g * CHUNK
                pltpu.sync_copy(dst_hbm.at[pl.ds(base, CHUNK)], dvec)
                pltpu.sync_copy(src_hbm.at[pl.ds(base, CHUNK)], svec)
                for j in range(CHUNK // LN):
                    d16 = dvec[pl.ds(j * LN, LN)]
                    msk = (d16 >= lo_v) & (d16 < hi_v)
                    lidx[pl.ds(j * LN, LN)] = jnp.where(
                        msk, d16 - lo_v, jnp.int32(RANGE))
                pltpu.sync_copy(xws_hbm.at[svec], rows)
                pltpu.sync_copy(rows, acc.at[lidx], add=True)

            @pl.loop(0, SEG_BASE)
            def _(i):
                chunk(sid * SEG_BASE + i)

            @pl.when(sid < SEG_EXTRA)
            def _():
                chunk(NS * SEG_BASE + sid)

            plsc.subcore_barrier()
            pltpu.sync_copy(
                acc.at[pl.ds(sid * STRIPE, STRIPE)],
                out_hbm.at[pl.ds(lo + sid * STRIPE, STRIPE)],
            )

    return k(xws, src, dst, zeros_acc)


BLK = 2000


def _encoder_tc(x, We1, be1, We2, be2):
    def body(x_ref, w1_ref, b1_ref, w2_ref, b2_ref, o_ref):
        h1 = jnp.dot(x_ref[...], w1_ref[...], preferred_element_type=jnp.float32)
        h1 = jnp.maximum(h1 + b1_ref[...], 0.0)
        h2 = jnp.dot(h1, w2_ref[...], preferred_element_type=jnp.float32)
        o_ref[...] = jnp.maximum(h2 + b2_ref[...], 0.0)

    return pl.pallas_call(
        body,
        grid=(N // BLK,),
        in_specs=[
            pl.BlockSpec((BLK, 32), lambda i: (i, 0)),
            pl.BlockSpec((32, 256), lambda i: (0, 0)),
            pl.BlockSpec((1, 256), lambda i: (0, 0)),
            pl.BlockSpec((256, 128), lambda i: (0, 0)),
            pl.BlockSpec((1, 128), lambda i: (0, 0)),
        ],
        out_specs=pl.BlockSpec((BLK, 128), lambda i: (i, 0)),
        out_shape=jax.ShapeDtypeStruct((N, 128), jnp.float32),
    )(x, We1, be1[None, :], We2, be2[None, :])


def _dinv_tc(degw):
    def body(d_ref, o_ref):
        d = d_ref[...]
        o_ref[...] = lax.rsqrt(d[0, :, 0:1] + d[1, :, 0:1] + 1.0)

    return pl.pallas_call(
        body,
        grid=(N // BLK,),
        in_specs=[pl.BlockSpec((NC, BLK, LN), lambda i: (0, i, 0))],
        out_specs=pl.BlockSpec((BLK, 1), lambda i: (i, 0)),
        out_shape=jax.ShapeDtypeStruct((N, 1), jnp.float32),
    )(degw)


def _pre_tc(h, W, dinv):
    din = h.shape[1]

    def body(h_ref, w_ref, d_ref, o_ref):
        xw = jnp.dot(h_ref[...], w_ref[...], preferred_element_type=jnp.float32)
        o_ref[...] = xw * d_ref[...]

    return pl.pallas_call(
        body,
        grid=(N // BLK,),
        in_specs=[
            pl.BlockSpec((BLK, din), lambda i: (i, 0)),
            pl.BlockSpec((din, F), lambda i: (0, 0)),
            pl.BlockSpec((BLK, 1), lambda i: (i, 0)),
        ],
        out_specs=pl.BlockSpec((BLK, F), lambda i: (i, 0)),
        out_shape=jax.ShapeDtypeStruct((N, F), jnp.float32),
    )(h, W, dinv)


def _post_tc(S_pad, xws, dinv, b, res):
    def body_res(s_ref, xws_ref, d_ref, b_ref, r_ref, o_ref):
        v = d_ref[...] * (s_ref[...] + xws_ref[...]) + b_ref[...]
        o_ref[...] = r_ref[...] + jnp.maximum(v, 0.0)

    def body_nores(s_ref, xws_ref, d_ref, b_ref, o_ref):
        v = d_ref[...] * (s_ref[...] + xws_ref[...]) + b_ref[...]
        o_ref[...] = jnp.maximum(v, 0.0)

    in_specs = [
        pl.BlockSpec((BLK, F), lambda i: (i, 0)),
        pl.BlockSpec((BLK, F), lambda i: (i, 0)),
        pl.BlockSpec((BLK, 1), lambda i: (i, 0)),
        pl.BlockSpec((1, F), lambda i: (0, 0)),
    ]
    args = [S_pad, xws, dinv, b[None, :]]
    body = body_nores
    if res is not None:
        in_specs.append(pl.BlockSpec((BLK, F), lambda i: (i, 0)))
        args.append(res)
        body = body_res

    return pl.pallas_call(
        body,
        grid=(N // BLK,),
        in_specs=in_specs,
        out_specs=pl.BlockSpec((BLK, F), lambda i: (i, 0)),
        out_shape=jax.ShapeDtypeStruct((N, F), jnp.float32),
    )(*args)


def kernel(x, edge_index, We1, be1, We2, be2, W0, b0, W1, b1, W2, b2):
    src = edge_index[0].astype(jnp.int32)
    dst = edge_index[1].astype(jnp.int32)

    ones_tile = jnp.ones((CHUNK, LN), jnp.float32)
    zeros_deg = jnp.zeros((DEG_STRIPE, LN), jnp.float32)
    zeros_acc = jnp.zeros((STRIPE, F), jnp.float32)

    degw = _deg_sc(dst, ones_tile, zeros_deg)
    h = _encoder_tc(x, We1, be1, We2, be2)
    dinv = _dinv_tc(degw)

    for W, b, has_res in ((W0, b0, False), (W1, b1, True), (W2, b2, True)):
        xws = _pre_tc(h, W, dinv)
        S_pad = _segsum_sc(xws, src, dst, zeros_acc)
        h = _post_tc(S_pad, xws, dinv, b, h if has_res else None)

    return h
